# baseline (speedup 1.0000x reference)
import jax
import jax.numpy as jnp
from jax import lax
from jax.experimental import pallas as pl
from jax.experimental.pallas import tpu as pltpu
from jax.experimental.pallas import tpu_sc as plsc

N = 100000
K = 256
L = 16
NW = 16
NC = 2
CHUNK = 6272
NPAD = NW * CHUNK
VREGS = CHUNK // L
CAND = 512
LN2 = 0.6931471805599453


def _iota():
    return lax.iota(jnp.int32, L)


def _scalar_at(vec, lane):
    return jnp.max(jnp.where(_iota() == lane, vec, jnp.zeros_like(vec)))


def _suffix_counts(hvreg):
    return lax.rev(plsc.cumsum(lax.rev(hvreg, (0,))), (0,))


def _body(keys_hbm, lab_hbm, disc_hbm, out_hbm,
          keys_v, lab_v, disc_v, hist_v, allhist_v, stripe_v,
          cand_v, candpay_v, allcand_v, allpay_v, gbuf_v, gpay_v,
          cnt_v, part_v,
          sh_hist, sh_merged, sh_cnt, sh_cand, sh_pay, sh_g, sh_gpay,
          sh_part):
    cid = lax.axis_index("c")
    wid = lax.axis_index("s")
    base = wid * CHUNK

    pltpu.sync_copy(keys_hbm.at[cid, pl.ds(base, CHUNK)], keys_v)
    pltpu.sync_copy(lab_hbm.at[pl.ds(base, CHUNK)], lab_v)
    pltpu.sync_copy(disc_hbm, disc_v)

    zeros_i = jnp.zeros((L,), jnp.int32)
    ones_i = jnp.ones((L,), jnp.int32)

    shoff = cid * NW * 256
    shoff32 = cid * NW * 32

    prefix = jnp.int32(0)
    krem = jnp.int32(K)
    for p in range(4):
        shift = 24 - 8 * p

        def zero_body(g, c):
            hist_v[pl.ds(g * L, L)] = zeros_i
            return c
        lax.fori_loop(0, 16, zero_body, jnp.int32(0))

        if p == 0:
            def scan_body(j, carry):
                for q in range(2):
                    k = keys_v[pl.ds(j * 32 + q * L, L)]
                    d = lax.shift_right_logical(k, 24) ^ 0x80
                    plsc.addupdate_scatter(hist_v, [d], ones_i)
                return carry
            lax.fori_loop(0, VREGS // 2, scan_body, jnp.int32(0))
        else:
            high_mask = jnp.int32(-(1 << (shift + 8)))
            pref_s = prefix ^ jnp.int32(-(1 << 31))

            def scan_body(j, carry):
                pp = carry
                for q in range(2):
                    k = keys_v[pl.ds(j * 32 + q * L, L)]
                    m = (k & high_mask) == pp
                    d = lax.shift_right_logical(k, shift) & 0xFF
                    plsc.addupdate_scatter(hist_v, [d], ones_i, mask=m)
                return pp
            lax.fori_loop(0, VREGS // 2, scan_body, pref_s)

        pltpu.sync_copy(hist_v, sh_hist.at[pl.ds(shoff + wid * 256, 256)])
        plsc.subcore_barrier()
        mybins = wid * L
        pltpu.sync_copy(sh_hist.at[pl.ds(shoff, NW * 256)], allhist_v)

        def sum_w(w, acc):
            return acc + allhist_v[pl.ds(w * 256 + mybins, L)]
        s0 = lax.fori_loop(0, NW, sum_w, zeros_i)
        stripe_v[pl.ds(0, L)] = s0
        pltpu.sync_copy(stripe_v.at[pl.ds(0, L)],
                        sh_merged.at[pl.ds(cid * 256 + mybins, L)])
        plsc.subcore_barrier()
        pltpu.sync_copy(sh_merged.at[pl.ds(cid * 256, 256)], hist_v)

        bs = [jnp.sum(hist_v[pl.ds(g * L, L)]) for g in range(16)]
        sb = [jnp.int32(0)] * 16
        run = jnp.int32(0)
        for g in range(15, -1, -1):
            sb[g] = run
            run = run + bs[g]
        t = jnp.int32(-1)
        for g in range(16):
            h = hist_v[pl.ds(g * L, L)]
            s = _suffix_counts(h) + sb[g]
            digs = _iota() + (g * L)
            c = jnp.where(s >= krem, digs, jnp.full((L,), -1, jnp.int32))
            t = jnp.maximum(t, jnp.max(c))
        above = jnp.int32(0)
        for g in range(16):
            h = hist_v[pl.ds(g * L, L)]
            digs = _iota() + (g * L)
            above = above + jnp.sum(jnp.where(digs > t, h, zeros_i))
        krem = krem - above
        prefix = prefix | lax.shift_left(t, shift)
        plsc.subcore_barrier()

    thr = prefix ^ jnp.int32(-(1 << 31))

    def compact_body(j, carry):
        cp = carry
        k = keys_v[pl.ds(j * L, L)]
        lab = lab_v[pl.ds(j * L, L)]
        gidx = base + j * L + _iota()
        m = k >= thr
        pc = plsc.cumsum(jnp.where(m, ones_i, zeros_i))
        d = jnp.minimum(cp + pc - 1, CAND - 1)
        plsc.store_scatter(cand_v, [d], k, mask=m)
        plsc.store_scatter(cand_v, [d + 512], gidx, mask=m)
        plsc.store_scatter(candpay_v, [d], lab, mask=m)
        return cp + plsc.all_reduce_population_count(m)

    cntv = lax.fori_loop(0, VREGS, compact_body, zeros_i)
    cnt = _scalar_at(cntv, 0)

    cnt_v[pl.ds(0, L)] = cntv
    pltpu.sync_copy(cnt_v.at[pl.ds(0, L)],
                    sh_cnt.at[pl.ds(shoff32 + wid * L, L)])
    pltpu.sync_copy(cand_v, sh_cand.at[pl.ds(cid * NW * 1024 + wid * 1024, 1024)])
    pltpu.sync_copy(candpay_v, sh_pay.at[pl.ds(cid * NW * 512 + wid * 512, 512)])
    plsc.subcore_barrier()

    @pl.when(wid == 0)
    def _compact_global():
        pltpu.sync_copy(sh_cnt.at[pl.ds(shoff32, NW * L)], cnt_v)
        pltpu.sync_copy(sh_cand.at[pl.ds(cid * NW * 1024, NW * 1024)],
                        allcand_v)
        pltpu.sync_copy(sh_pay.at[pl.ds(cid * NW * 512, NW * 512)], allpay_v)

        def zero_g(g, c):
            gbuf_v[pl.ds(g * L, L)] = zeros_i
            return c
        lax.fori_loop(0, 1024 // L, zero_g, jnp.int32(0))

        def zero_p(g, c):
            gpay_v[pl.ds(g * L, L)] = jnp.zeros((L,), jnp.float32)
            return c
        lax.fori_loop(0, 512 // L, zero_p, jnp.int32(0))

        off = jnp.int32(0)
        for w in range(NW):
            cw = _scalar_at(cnt_v[pl.ds(w * L, L)], 0)

            def copy_body(i, o):
                lanes = i * L + _iota()
                m = lanes < cw
                d = jnp.minimum(o + lanes, CAND - 1)
                kk = allcand_v[pl.ds(w * 1024 + i * L, L)]
                ii = allcand_v[pl.ds(w * 1024 + 512 + i * L, L)]
                pp = allpay_v[pl.ds(w * 512 + i * L, L)]
                plsc.store_scatter(gbuf_v, [d], kk, mask=m)
                plsc.store_scatter(gbuf_v, [d + 512], ii, mask=m)
                plsc.store_scatter(gpay_v, [d], pp, mask=m)
                return o
            trips = lax.div(cw + (L - 1), jnp.int32(L))
            lax.fori_loop(0, trips, copy_body, off)
            off = jnp.minimum(off + cw, jnp.int32(CAND))
        pltpu.sync_copy(gbuf_v, sh_g.at[pl.ds(cid * 1024, 1024)])
        pltpu.sync_copy(gpay_v, sh_gpay.at[pl.ds(cid * 512, 512)])
    plsc.subcore_barrier()

    pltpu.sync_copy(sh_g.at[pl.ds(cid * 1024, 1024)], gbuf_v)
    pltpu.sync_copy(sh_gpay.at[pl.ds(cid * 512, 512)], gpay_v)
    pltpu.sync_copy(sh_cnt.at[pl.ds(shoff32, NW * L)], cnt_v)

    def sum_cnt(w, acc):
        return acc + cnt_v[pl.ds(w * L, L)]
    cmax = jnp.minimum(_scalar_at(lax.fori_loop(0, NW, sum_cnt, zeros_i), 0),
                       jnp.int32(CAND))
    mybase = wid * 32

    mk0 = gbuf_v[pl.ds(mybase, L)]
    mk1 = gbuf_v[pl.ds(mybase + L, L)]
    mi0 = gbuf_v[pl.ds(512 + mybase, L)]
    mi1 = gbuf_v[pl.ds(512 + mybase + L, L)]

    def rank_body(j, carry):
        r0, r1 = carry
        jv = jnp.full((L,), j, jnp.int32)
        bk = plsc.load_gather(gbuf_v, [jv])
        bi = plsc.load_gather(gbuf_v, [jv + 512])
        r0 = r0 + jnp.where((bk > mk0) | ((bk == mk0) & (bi < mi0)),
                            ones_i, zeros_i)
        r1 = r1 + jnp.where((bk > mk1) | ((bk == mk1) & (bi < mi1)),
                            ones_i, zeros_i)
        return r0, r1

    r0, r1 = lax.fori_loop(0, cmax, rank_body, (zeros_i, zeros_i))

    g0 = jnp.exp(gpay_v[pl.ds(mybase, L)] * LN2) - 1.0
    g1 = jnp.exp(gpay_v[pl.ds(mybase + L, L)] * LN2) - 1.0
    d0 = plsc.load_gather(disc_v, [r0])
    d1 = plsc.load_gather(disc_v, [r1])
    part_v[pl.ds(0, L)] = g0 * d0 + g1 * d1
    pltpu.sync_copy(part_v.at[pl.ds(0, L)],
                    sh_part.at[pl.ds(cid * NW * L + wid * L, L)])
    plsc.subcore_barrier()

    @pl.when(wid == 0)
    def _finish():
        pltpu.sync_copy(sh_part.at[pl.ds(cid * NW * L, NW * L)],
                        allpay_v.at[pl.ds(0, NW * L)])

        def red(w, acc):
            return acc + allpay_v[pl.ds(w * L, L)]
        tot = lax.fori_loop(0, NW, red, jnp.zeros((L,), jnp.float32))
        part_v[pl.ds(0, L)] = jnp.full((L,), jnp.sum(tot), jnp.float32)
        pltpu.sync_copy(part_v.at[pl.ds(0, L)], out_hbm.at[cid])


def kernel(preds, labels):
    preds_p = jnp.concatenate(
        [preds, jnp.full((NPAD - N,), -jnp.inf, jnp.float32)])
    labels_p = jnp.concatenate(
        [labels, jnp.full((NPAD - N,), -jnp.inf, jnp.float32)])
    lab_pay = jnp.concatenate([labels, jnp.zeros((NPAD - N,), jnp.float32)])

    def skey(x):
        u = lax.bitcast_convert_type(x, jnp.uint32)
        s = jnp.where(u >> 31 == 1, ~u, u | jnp.uint32(0x80000000))
        return lax.bitcast_convert_type(s ^ jnp.uint32(0x80000000), jnp.int32)

    keys = jnp.stack([skey(preds_p), skey(labels_p)])
    disc = jnp.concatenate([
        1.0 / jnp.log2(jnp.arange(K, dtype=jnp.float32) + 2.0),
        jnp.zeros((CAND - K,), jnp.float32)])

    mesh = plsc.VectorSubcoreMesh(core_axis_name="c", subcore_axis_name="s",
                                  num_cores=NC)
    k = pl.kernel(
        _body,
        out_type=jax.ShapeDtypeStruct((NC, L), jnp.float32),
        mesh=mesh,
        compiler_params=pltpu.CompilerParams(needs_layout_passes=False),
        scratch_types=[
            pltpu.VMEM((CHUNK,), jnp.int32),
            pltpu.VMEM((CHUNK,), jnp.float32),
            pltpu.VMEM((CAND,), jnp.float32),
            pltpu.VMEM((256,), jnp.int32),
            pltpu.VMEM((NW * 256,), jnp.int32),
            pltpu.VMEM((L,), jnp.int32),
            pltpu.VMEM((1024,), jnp.int32),
            pltpu.VMEM((512,), jnp.float32),
            pltpu.VMEM((NW * 1024,), jnp.int32),
            pltpu.VMEM((NW * 512,), jnp.float32),
            pltpu.VMEM((1024,), jnp.int32),
            pltpu.VMEM((512,), jnp.float32),
            pltpu.VMEM((NW * L,), jnp.int32),
            pltpu.VMEM((L,), jnp.float32),
            pltpu.VMEM_SHARED((NC * NW * 256,), jnp.int32),
            pltpu.VMEM_SHARED((NC * 256,), jnp.int32),
            pltpu.VMEM_SHARED((NC * NW * L,), jnp.int32),
            pltpu.VMEM_SHARED((NC * NW * 1024,), jnp.int32),
            pltpu.VMEM_SHARED((NC * NW * 512,), jnp.float32),
            pltpu.VMEM_SHARED((NC * 1024,), jnp.int32),
            pltpu.VMEM_SHARED((NC * 512,), jnp.float32),
            pltpu.VMEM_SHARED((NC * NW * L,), jnp.float32),
        ],
    )
    out = k(keys, lab_pay, disc)
    dcg = out[0, 0]
    idcg = out[1, 0]
    ndcg = jnp.where(idcg == 0.0, 0.0, dcg / idcg)
    return 1.0 - ndcg


# speedup vs baseline: 1.0123x; 1.0123x over previous
import jax
import jax.numpy as jnp
from jax import lax
from jax.experimental import pallas as pl
from jax.experimental.pallas import tpu as pltpu
from jax.experimental.pallas import tpu_sc as plsc

N = 100000
K = 256
L = 16
NW = 16
NC = 2
CHUNK = 6272
NPAD = NW * CHUNK
VREGS = CHUNK // L
CAND = 512
LN2 = 0.6931471805599453


def _iota():
    return lax.iota(jnp.int32, L)


def _scalar_at(vec, lane):
    return jnp.max(jnp.where(_iota() == lane, vec, jnp.zeros_like(vec)))


def _suffix_counts(hvreg):
    return lax.rev(plsc.cumsum(lax.rev(hvreg, (0,))), (0,))


def _body(keys_hbm, lab_hbm, disc_hbm, out_hbm,
          keys_v, lab_v, disc_v, hist_v, allhist_v, stripe_v,
          cand_v, candpay_v, allcand_v, allpay_v, gbuf_v, gpay_v,
          cnt_v, part_v,
          sh_hist, sh_merged, sh_cnt, sh_cand, sh_pay, sh_g, sh_gpay,
          sh_part):
    cid = lax.axis_index("c")
    wid = lax.axis_index("s")
    base = wid * CHUNK

    pltpu.sync_copy(keys_hbm.at[cid, pl.ds(base, CHUNK)], keys_v)
    pltpu.sync_copy(lab_hbm.at[pl.ds(base, CHUNK)], lab_v)
    pltpu.sync_copy(disc_hbm, disc_v)

    zeros_i = jnp.zeros((L,), jnp.int32)
    ones_i = jnp.ones((L,), jnp.int32)

    shoff = cid * NW * 256
    shoff32 = cid * NW * 32

    prefix = jnp.int32(0)
    krem = jnp.int32(K)
    for p in range(4):
        shift = 24 - 8 * p

        def zero_body(g, c):
            hist_v[pl.ds(g * L, L)] = zeros_i
            return c
        lax.fori_loop(0, 16, zero_body, jnp.int32(0))

        if p == 0:
            def scan_body(j, carry):
                for q in range(2):
                    k = keys_v[pl.ds(j * 32 + q * L, L)]
                    d = lax.shift_right_logical(k, 24) ^ 0x80
                    plsc.addupdate_scatter(hist_v, [d], ones_i)
                return carry
            lax.fori_loop(0, VREGS // 2, scan_body, jnp.int32(0))
        else:
            high_mask = jnp.int32(-(1 << (shift + 8)))
            pref_s = prefix ^ jnp.int32(-(1 << 31))

            def scan_body(j, carry):
                pp = carry
                for q in range(2):
                    k = keys_v[pl.ds(j * 32 + q * L, L)]
                    m = (k & high_mask) == pp
                    d = lax.shift_right_logical(k, shift) & 0xFF
                    plsc.addupdate_scatter(hist_v, [d], ones_i, mask=m)
                return pp
            lax.fori_loop(0, VREGS // 2, scan_body, pref_s)

        pltpu.sync_copy(hist_v, sh_hist.at[pl.ds(shoff + wid * 256, 256)])
        plsc.subcore_barrier()
        mybins = wid * L
        pltpu.sync_copy(sh_hist.at[pl.ds(shoff, NW * 256)], allhist_v)

        def sum_w(w, acc):
            return acc + allhist_v[pl.ds(w * 256 + mybins, L)]
        s0 = lax.fori_loop(0, NW, sum_w, zeros_i)
        stripe_v[pl.ds(0, L)] = s0
        pltpu.sync_copy(stripe_v.at[pl.ds(0, L)],
                        sh_merged.at[pl.ds(cid * 256 + mybins, L)])
        plsc.subcore_barrier()
        pltpu.sync_copy(sh_merged.at[pl.ds(cid * 256, 256)], hist_v)

        bs = [jnp.sum(hist_v[pl.ds(g * L, L)]) for g in range(16)]
        sb = [jnp.int32(0)] * 16
        run = jnp.int32(0)
        for g in range(15, -1, -1):
            sb[g] = run
            run = run + bs[g]
        t = jnp.int32(-1)
        for g in range(16):
            h = hist_v[pl.ds(g * L, L)]
            s = _suffix_counts(h) + sb[g]
            digs = _iota() + (g * L)
            c = jnp.where(s >= krem, digs, jnp.full((L,), -1, jnp.int32))
            t = jnp.maximum(t, jnp.max(c))
        above = jnp.int32(0)
        for g in range(16):
            h = hist_v[pl.ds(g * L, L)]
            digs = _iota() + (g * L)
            above = above + jnp.sum(jnp.where(digs > t, h, zeros_i))
        krem = krem - above
        prefix = prefix | lax.shift_left(t, shift)
        plsc.subcore_barrier()

    thr = prefix ^ jnp.int32(-(1 << 31))

    def compact_body(j, carry):
        cp = carry
        for q in range(2):
            k = keys_v[pl.ds(j * 32 + q * L, L)]
            lab = lab_v[pl.ds(j * 32 + q * L, L)]
            gidx = base + j * 32 + q * L + _iota()
            m = k >= thr
            pc = plsc.cumsum(jnp.where(m, ones_i, zeros_i))
            d = jnp.minimum(cp + pc - 1, CAND - 1)
            plsc.store_scatter(cand_v, [d], k, mask=m)
            plsc.store_scatter(cand_v, [d + 512], gidx, mask=m)
            plsc.store_scatter(candpay_v, [d], lab, mask=m)
            cp = cp + plsc.all_reduce_population_count(m)
        return cp

    cntv = lax.fori_loop(0, VREGS // 2, compact_body, zeros_i)
    cnt = _scalar_at(cntv, 0)

    cnt_v[pl.ds(0, L)] = cntv
    pltpu.sync_copy(cnt_v.at[pl.ds(0, L)],
                    sh_cnt.at[pl.ds(shoff32 + wid * L, L)])
    pltpu.sync_copy(cand_v, sh_cand.at[pl.ds(cid * NW * 1024 + wid * 1024, 1024)])
    pltpu.sync_copy(candpay_v, sh_pay.at[pl.ds(cid * NW * 512 + wid * 512, 512)])
    plsc.subcore_barrier()

    @pl.when(wid == 0)
    def _compact_global():
        pltpu.sync_copy(sh_cnt.at[pl.ds(shoff32, NW * L)], cnt_v)
        pltpu.sync_copy(sh_cand.at[pl.ds(cid * NW * 1024, NW * 1024)],
                        allcand_v)
        pltpu.sync_copy(sh_pay.at[pl.ds(cid * NW * 512, NW * 512)], allpay_v)

        def zero_g(g, c):
            gbuf_v[pl.ds(g * L, L)] = zeros_i
            return c
        lax.fori_loop(0, 1024 // L, zero_g, jnp.int32(0))

        def zero_p(g, c):
            gpay_v[pl.ds(g * L, L)] = jnp.zeros((L,), jnp.float32)
            return c
        lax.fori_loop(0, 512 // L, zero_p, jnp.int32(0))

        off = jnp.int32(0)
        for w in range(NW):
            cw = _scalar_at(cnt_v[pl.ds(w * L, L)], 0)

            def copy_body(i, o):
                lanes = i * L + _iota()
                m = lanes < cw
                d = jnp.minimum(o + lanes, CAND - 1)
                kk = allcand_v[pl.ds(w * 1024 + i * L, L)]
                ii = allcand_v[pl.ds(w * 1024 + 512 + i * L, L)]
                pp = allpay_v[pl.ds(w * 512 + i * L, L)]
                plsc.store_scatter(gbuf_v, [d], kk, mask=m)
                plsc.store_scatter(gbuf_v, [d + 512], ii, mask=m)
                plsc.store_scatter(gpay_v, [d], pp, mask=m)
                return o
            trips = lax.div(cw + (L - 1), jnp.int32(L))
            lax.fori_loop(0, trips, copy_body, off)
            off = jnp.minimum(off + cw, jnp.int32(CAND))
        pltpu.sync_copy(gbuf_v, sh_g.at[pl.ds(cid * 1024, 1024)])
        pltpu.sync_copy(gpay_v, sh_gpay.at[pl.ds(cid * 512, 512)])
    plsc.subcore_barrier()

    pltpu.sync_copy(sh_g.at[pl.ds(cid * 1024, 1024)], gbuf_v)
    pltpu.sync_copy(sh_gpay.at[pl.ds(cid * 512, 512)], gpay_v)
    pltpu.sync_copy(sh_cnt.at[pl.ds(shoff32, NW * L)], cnt_v)

    def sum_cnt(w, acc):
        return acc + cnt_v[pl.ds(w * L, L)]
    cmax = jnp.minimum(_scalar_at(lax.fori_loop(0, NW, sum_cnt, zeros_i), 0),
                       jnp.int32(CAND))
    mybase = wid * 32

    mk0 = gbuf_v[pl.ds(mybase, L)]
    mk1 = gbuf_v[pl.ds(mybase + L, L)]
    mi0 = gbuf_v[pl.ds(512 + mybase, L)]
    mi1 = gbuf_v[pl.ds(512 + mybase + L, L)]

    def rank_body(jg, carry):
        r0, r1 = carry
        for q in range(2):
            jv = jnp.full((L,), jg * 2 + q, jnp.int32)
            bk = plsc.load_gather(gbuf_v, [jv])
            bi = plsc.load_gather(gbuf_v, [jv + 512])
            r0 = r0 + jnp.where((bk > mk0) | ((bk == mk0) & (bi < mi0)),
                                ones_i, zeros_i)
            r1 = r1 + jnp.where((bk > mk1) | ((bk == mk1) & (bi < mi1)),
                                ones_i, zeros_i)
        return r0, r1

    r0, r1 = lax.fori_loop(0, lax.div(cmax + 1, jnp.int32(2)), rank_body,
                           (zeros_i, zeros_i))

    g0 = jnp.exp(gpay_v[pl.ds(mybase, L)] * LN2) - 1.0
    g1 = jnp.exp(gpay_v[pl.ds(mybase + L, L)] * LN2) - 1.0
    d0 = plsc.load_gather(disc_v, [r0])
    d1 = plsc.load_gather(disc_v, [r1])
    part_v[pl.ds(0, L)] = g0 * d0 + g1 * d1
    pltpu.sync_copy(part_v.at[pl.ds(0, L)],
                    sh_part.at[pl.ds(cid * NW * L + wid * L, L)])
    plsc.subcore_barrier()

    @pl.when(wid == 0)
    def _finish():
        pltpu.sync_copy(sh_part.at[pl.ds(cid * NW * L, NW * L)],
                        allpay_v.at[pl.ds(0, NW * L)])

        def red(w, acc):
            return acc + allpay_v[pl.ds(w * L, L)]
        tot = lax.fori_loop(0, NW, red, jnp.zeros((L,), jnp.float32))
        part_v[pl.ds(0, L)] = jnp.full((L,), jnp.sum(tot), jnp.float32)
        pltpu.sync_copy(part_v.at[pl.ds(0, L)], out_hbm.at[cid])


def kernel(preds, labels):
    preds_p = jnp.concatenate(
        [preds, jnp.full((NPAD - N,), -jnp.inf, jnp.float32)])
    labels_p = jnp.concatenate(
        [labels, jnp.full((NPAD - N,), -jnp.inf, jnp.float32)])
    lab_pay = jnp.concatenate([labels, jnp.zeros((NPAD - N,), jnp.float32)])

    def skey(x):
        u = lax.bitcast_convert_type(x, jnp.uint32)
        s = jnp.where(u >> 31 == 1, ~u, u | jnp.uint32(0x80000000))
        return lax.bitcast_convert_type(s ^ jnp.uint32(0x80000000), jnp.int32)

    keys = jnp.stack([skey(preds_p), skey(labels_p)])
    disc = jnp.concatenate([
        1.0 / jnp.log2(jnp.arange(K, dtype=jnp.float32) + 2.0),
        jnp.zeros((CAND - K,), jnp.float32)])

    mesh = plsc.VectorSubcoreMesh(core_axis_name="c", subcore_axis_name="s",
                                  num_cores=NC)
    k = pl.kernel(
        _body,
        out_type=jax.ShapeDtypeStruct((NC, L), jnp.float32),
        mesh=mesh,
        compiler_params=pltpu.CompilerParams(needs_layout_passes=False),
        scratch_types=[
            pltpu.VMEM((CHUNK,), jnp.int32),
            pltpu.VMEM((CHUNK,), jnp.float32),
            pltpu.VMEM((CAND,), jnp.float32),
            pltpu.VMEM((256,), jnp.int32),
            pltpu.VMEM((NW * 256,), jnp.int32),
            pltpu.VMEM((L,), jnp.int32),
            pltpu.VMEM((1024,), jnp.int32),
            pltpu.VMEM((512,), jnp.float32),
            pltpu.VMEM((NW * 1024,), jnp.int32),
            pltpu.VMEM((NW * 512,), jnp.float32),
            pltpu.VMEM((1024,), jnp.int32),
            pltpu.VMEM((512,), jnp.float32),
            pltpu.VMEM((NW * L,), jnp.int32),
            pltpu.VMEM((L,), jnp.float32),
            pltpu.VMEM_SHARED((NC * NW * 256,), jnp.int32),
            pltpu.VMEM_SHARED((NC * 256,), jnp.int32),
            pltpu.VMEM_SHARED((NC * NW * L,), jnp.int32),
            pltpu.VMEM_SHARED((NC * NW * 1024,), jnp.int32),
            pltpu.VMEM_SHARED((NC * NW * 512,), jnp.float32),
            pltpu.VMEM_SHARED((NC * 1024,), jnp.int32),
            pltpu.VMEM_SHARED((NC * 512,), jnp.float32),
            pltpu.VMEM_SHARED((NC * NW * L,), jnp.float32),
        ],
    )
    out = k(keys, lab_pay, disc)
    dcg = out[0, 0]
    idcg = out[1, 0]
    ndcg = jnp.where(idcg == 0.0, 0.0, dcg / idcg)
    return 1.0 - ndcg


# speedup vs baseline: 1.0366x; 1.0239x over previous
import jax
import jax.numpy as jnp
from jax import lax
from jax.experimental import pallas as pl
from jax.experimental.pallas import tpu as pltpu
from jax.experimental.pallas import tpu_sc as plsc

N = 100000
K = 256
L = 16
NW = 16
NC = 2
CHUNK = 6272
NPAD = NW * CHUNK
VREGS = CHUNK // L
CAND = 512
LN2 = 0.6931471805599453


def _iota():
    return lax.iota(jnp.int32, L)


def _scalar_at(vec, lane):
    return jnp.max(jnp.where(_iota() == lane, vec, jnp.zeros_like(vec)))


def _suffix_counts(hvreg):
    return lax.rev(plsc.cumsum(lax.rev(hvreg, (0,))), (0,))


def _body(keys_hbm, lab_hbm, disc_hbm, out_hbm,
          keys_v, lab_v, disc_v, hist_v, allhist_v, stripe_v,
          cand_v, candpay_v, allcand_v, allpay_v, gbuf_v, gpay_v,
          cnt_v, part_v, sem,
          sh_hist, sh_merged, sh_cnt, sh_cand, sh_pay, sh_g, sh_gpay,
          sh_part):
    cid = lax.axis_index("c")
    wid = lax.axis_index("s")
    base = wid * CHUNK

    c1 = pltpu.async_copy(keys_hbm.at[cid, pl.ds(base, CHUNK)], keys_v, sem)
    c2 = pltpu.async_copy(lab_hbm.at[pl.ds(base, CHUNK)], lab_v, sem)
    c3 = pltpu.async_copy(disc_hbm, disc_v, sem)
    c1.wait()
    c2.wait()
    c3.wait()

    zeros_i = jnp.zeros((L,), jnp.int32)
    ones_i = jnp.ones((L,), jnp.int32)

    shoff = cid * NW * 256
    shoff32 = cid * NW * 32

    prefix = jnp.int32(0)
    krem = jnp.int32(K)
    for p in range(4):
        shift = 24 - 8 * p

        def zero_body(g, c):
            hist_v[pl.ds(g * L, L)] = zeros_i
            return c
        lax.fori_loop(0, 16, zero_body, jnp.int32(0))

        if p == 0:
            def scan_body(j, carry):
                for q in range(2):
                    k = keys_v[pl.ds(j * 32 + q * L, L)]
                    d = lax.shift_right_logical(k, 24) ^ 0x80
                    plsc.addupdate_scatter(hist_v, [d], ones_i)
                return carry
            lax.fori_loop(0, VREGS // 2, scan_body, jnp.int32(0))
        else:
            high_mask = jnp.int32(-(1 << (shift + 8)))
            pref_s = prefix ^ jnp.int32(-(1 << 31))

            def scan_body(j, carry):
                pp = carry
                for q in range(2):
                    k = keys_v[pl.ds(j * 32 + q * L, L)]
                    m = (k & high_mask) == pp
                    d = lax.shift_right_logical(k, shift) & 0xFF
                    plsc.addupdate_scatter(hist_v, [d], ones_i, mask=m)
                return pp
            lax.fori_loop(0, VREGS // 2, scan_body, pref_s)

        pltpu.sync_copy(hist_v, sh_hist.at[pl.ds(shoff + wid * 256, 256)])
        plsc.subcore_barrier()
        mybins = wid * L
        pltpu.sync_copy(sh_hist.at[pl.ds(shoff, NW * 256)], allhist_v)

        def sum_w(w, acc):
            return acc + allhist_v[pl.ds(w * 256 + mybins, L)]
        s0 = lax.fori_loop(0, NW, sum_w, zeros_i)
        stripe_v[pl.ds(0, L)] = s0
        pltpu.sync_copy(stripe_v.at[pl.ds(0, L)],
                        sh_merged.at[pl.ds(cid * 256 + mybins, L)])
        plsc.subcore_barrier()
        pltpu.sync_copy(sh_merged.at[pl.ds(cid * 256, 256)], hist_v)

        bs = [jnp.sum(hist_v[pl.ds(g * L, L)]) for g in range(16)]
        sb = [jnp.int32(0)] * 16
        run = jnp.int32(0)
        for g in range(15, -1, -1):
            sb[g] = run
            run = run + bs[g]
        t = jnp.int32(-1)
        for g in range(16):
            h = hist_v[pl.ds(g * L, L)]
            s = _suffix_counts(h) + sb[g]
            digs = _iota() + (g * L)
            c = jnp.where(s >= krem, digs, jnp.full((L,), -1, jnp.int32))
            t = jnp.maximum(t, jnp.max(c))
        above = jnp.int32(0)
        for g in range(16):
            h = hist_v[pl.ds(g * L, L)]
            digs = _iota() + (g * L)
            above = above + jnp.sum(jnp.where(digs > t, h, zeros_i))
        krem = krem - above
        prefix = prefix | lax.shift_left(t, shift)
        plsc.subcore_barrier()

    thr = prefix ^ jnp.int32(-(1 << 31))

    def compact_body(j, carry):
        cp = carry
        for q in range(2):
            k = keys_v[pl.ds(j * 32 + q * L, L)]
            lab = lab_v[pl.ds(j * 32 + q * L, L)]
            gidx = base + j * 32 + q * L + _iota()
            m = k >= thr
            pc = plsc.cumsum(jnp.where(m, ones_i, zeros_i))
            d = jnp.minimum(cp + pc - 1, CAND - 1)
            plsc.store_scatter(cand_v, [d], k, mask=m)
            plsc.store_scatter(cand_v, [d + 512], gidx, mask=m)
            plsc.store_scatter(candpay_v, [d], lab, mask=m)
            cp = cp + plsc.all_reduce_population_count(m)
        return cp

    cntv = lax.fori_loop(0, VREGS // 2, compact_body, zeros_i)
    cnt = _scalar_at(cntv, 0)

    cnt_v[pl.ds(0, L)] = cntv
    c1 = pltpu.async_copy(cnt_v.at[pl.ds(0, L)],
                          sh_cnt.at[pl.ds(shoff32 + wid * L, L)], sem)
    c2 = pltpu.async_copy(
        cand_v, sh_cand.at[pl.ds(cid * NW * 1024 + wid * 1024, 1024)], sem)
    c3 = pltpu.async_copy(
        candpay_v, sh_pay.at[pl.ds(cid * NW * 512 + wid * 512, 512)], sem)
    c1.wait()
    c2.wait()
    c3.wait()
    plsc.subcore_barrier()

    @pl.when(wid == 0)
    def _compact_global():
        pltpu.sync_copy(sh_cnt.at[pl.ds(shoff32, NW * L)], cnt_v)
        pltpu.sync_copy(sh_cand.at[pl.ds(cid * NW * 1024, NW * 1024)],
                        allcand_v)
        pltpu.sync_copy(sh_pay.at[pl.ds(cid * NW * 512, NW * 512)], allpay_v)

        def zero_g(g, c):
            gbuf_v[pl.ds(g * L, L)] = zeros_i
            return c
        lax.fori_loop(0, 1024 // L, zero_g, jnp.int32(0))

        def zero_p(g, c):
            gpay_v[pl.ds(g * L, L)] = jnp.zeros((L,), jnp.float32)
            return c
        lax.fori_loop(0, 512 // L, zero_p, jnp.int32(0))

        off = jnp.int32(0)
        for w in range(NW):
            cw = _scalar_at(cnt_v[pl.ds(w * L, L)], 0)

            def copy_body(i, o):
                lanes = i * L + _iota()
                m = lanes < cw
                d = jnp.minimum(o + lanes, CAND - 1)
                kk = allcand_v[pl.ds(w * 1024 + i * L, L)]
                ii = allcand_v[pl.ds(w * 1024 + 512 + i * L, L)]
                pp = allpay_v[pl.ds(w * 512 + i * L, L)]
                plsc.store_scatter(gbuf_v, [d], kk, mask=m)
                plsc.store_scatter(gbuf_v, [d + 512], ii, mask=m)
                plsc.store_scatter(gpay_v, [d], pp, mask=m)
                return o
            trips = lax.div(cw + (L - 1), jnp.int32(L))
            lax.fori_loop(0, trips, copy_body, off)
            off = jnp.minimum(off + cw, jnp.int32(CAND))
        pltpu.sync_copy(gbuf_v, sh_g.at[pl.ds(cid * 1024, 1024)])
        pltpu.sync_copy(gpay_v, sh_gpay.at[pl.ds(cid * 512, 512)])
    plsc.subcore_barrier()

    c1 = pltpu.async_copy(sh_g.at[pl.ds(cid * 1024, 1024)], gbuf_v, sem)
    c2 = pltpu.async_copy(sh_gpay.at[pl.ds(cid * 512, 512)], gpay_v, sem)
    c3 = pltpu.async_copy(sh_cnt.at[pl.ds(shoff32, NW * L)], cnt_v, sem)
    c1.wait()
    c2.wait()
    c3.wait()

    def sum_cnt(w, acc):
        return acc + cnt_v[pl.ds(w * L, L)]
    cmax = jnp.minimum(_scalar_at(lax.fori_loop(0, NW, sum_cnt, zeros_i), 0),
                       jnp.int32(CAND))
    mybase = wid * 32

    mk0 = gbuf_v[pl.ds(mybase, L)]
    mk1 = gbuf_v[pl.ds(mybase + L, L)]
    mi0 = gbuf_v[pl.ds(512 + mybase, L)]
    mi1 = gbuf_v[pl.ds(512 + mybase + L, L)]

    def rank_body(jg, carry):
        r0, r1 = carry
        for q in range(2):
            jv = jnp.full((L,), jg * 2 + q, jnp.int32)
            bk = plsc.load_gather(gbuf_v, [jv])
            bi = plsc.load_gather(gbuf_v, [jv + 512])
            r0 = r0 + jnp.where((bk > mk0) | ((bk == mk0) & (bi < mi0)),
                                ones_i, zeros_i)
            r1 = r1 + jnp.where((bk > mk1) | ((bk == mk1) & (bi < mi1)),
                                ones_i, zeros_i)
        return r0, r1

    r0, r1 = lax.fori_loop(0, lax.div(cmax + 1, jnp.int32(2)), rank_body,
                           (zeros_i, zeros_i))

    g0 = jnp.exp(gpay_v[pl.ds(mybase, L)] * LN2) - 1.0
    g1 = jnp.exp(gpay_v[pl.ds(mybase + L, L)] * LN2) - 1.0
    d0 = plsc.load_gather(disc_v, [r0])
    d1 = plsc.load_gather(disc_v, [r1])
    part_v[pl.ds(0, L)] = g0 * d0 + g1 * d1
    pltpu.sync_copy(part_v.at[pl.ds(0, L)],
                    sh_part.at[pl.ds(cid * NW * L + wid * L, L)])
    plsc.subcore_barrier()

    @pl.when(wid == 0)
    def _finish():
        pltpu.sync_copy(sh_part.at[pl.ds(cid * NW * L, NW * L)],
                        allpay_v.at[pl.ds(0, NW * L)])

        def red(w, acc):
            return acc + allpay_v[pl.ds(w * L, L)]
        tot = lax.fori_loop(0, NW, red, jnp.zeros((L,), jnp.float32))
        part_v[pl.ds(0, L)] = jnp.full((L,), jnp.sum(tot), jnp.float32)
        pltpu.sync_copy(part_v.at[pl.ds(0, L)], out_hbm.at[cid])


def kernel(preds, labels):
    preds_p = jnp.concatenate(
        [preds, jnp.full((NPAD - N,), -jnp.inf, jnp.float32)])
    labels_p = jnp.concatenate(
        [labels, jnp.full((NPAD - N,), -jnp.inf, jnp.float32)])
    lab_pay = jnp.concatenate([labels, jnp.zeros((NPAD - N,), jnp.float32)])

    def skey(x):
        u = lax.bitcast_convert_type(x, jnp.uint32)
        s = jnp.where(u >> 31 == 1, ~u, u | jnp.uint32(0x80000000))
        return lax.bitcast_convert_type(s ^ jnp.uint32(0x80000000), jnp.int32)

    keys = jnp.stack([skey(preds_p), skey(labels_p)])
    disc = jnp.concatenate([
        1.0 / jnp.log2(jnp.arange(K, dtype=jnp.float32) + 2.0),
        jnp.zeros((CAND - K,), jnp.float32)])

    mesh = plsc.VectorSubcoreMesh(core_axis_name="c", subcore_axis_name="s",
                                  num_cores=NC)
    k = pl.kernel(
        _body,
        out_type=jax.ShapeDtypeStruct((NC, L), jnp.float32),
        mesh=mesh,
        compiler_params=pltpu.CompilerParams(needs_layout_passes=False),
        scratch_types=[
            pltpu.VMEM((CHUNK,), jnp.int32),
            pltpu.VMEM((CHUNK,), jnp.float32),
            pltpu.VMEM((CAND,), jnp.float32),
            pltpu.VMEM((256,), jnp.int32),
            pltpu.VMEM((NW * 256,), jnp.int32),
            pltpu.VMEM((L,), jnp.int32),
            pltpu.VMEM((1024,), jnp.int32),
            pltpu.VMEM((512,), jnp.float32),
            pltpu.VMEM((NW * 1024,), jnp.int32),
            pltpu.VMEM((NW * 512,), jnp.float32),
            pltpu.VMEM((1024,), jnp.int32),
            pltpu.VMEM((512,), jnp.float32),
            pltpu.VMEM((NW * L,), jnp.int32),
            pltpu.VMEM((L,), jnp.float32),
            pltpu.SemaphoreType.DMA,
            pltpu.VMEM_SHARED((NC * NW * 256,), jnp.int32),
            pltpu.VMEM_SHARED((NC * 256,), jnp.int32),
            pltpu.VMEM_SHARED((NC * NW * L,), jnp.int32),
            pltpu.VMEM_SHARED((NC * NW * 1024,), jnp.int32),
            pltpu.VMEM_SHARED((NC * NW * 512,), jnp.float32),
            pltpu.VMEM_SHARED((NC * 1024,), jnp.int32),
            pltpu.VMEM_SHARED((NC * 512,), jnp.float32),
            pltpu.VMEM_SHARED((NC * NW * L,), jnp.float32),
        ],
    )
    out = k(keys, lab_pay, disc)
    dcg = out[0, 0]
    idcg = out[1, 0]
    ndcg = jnp.where(idcg == 0.0, 0.0, dcg / idcg)
    return 1.0 - ndcg
